# baseline (speedup 1.0000x reference)
import functools

import jax
import jax.numpy as jnp
from jax import lax
from jax.experimental import pallas as pl
from jax.experimental.pallas import tpu as pltpu
from jax.experimental.pallas import tpu_sc as plsc

N = 10000
NP = 10240
E = 160000
HID = 512
OUTC = 128
NC = 2
NS = 16
LB = 128
NW = NC * NS
EP = ((E + NW * LB - 1) // (NW * LB)) * (NW * LB)
NB = EP // LB
BPW = NB // NW
RPT = NP // NS
F32 = jnp.float32
BM = 256


def _mesh():
    return plsc.VectorSubcoreMesh(core_axis_name="c", subcore_axis_name="s",
                                  num_cores=NC, num_subcores=NS)


_SC_CACHE = {}


def _lazy(name, builder):
    def wrapper(*args):
        if name not in _SC_CACHE:
            _SC_CACHE[name] = builder()
        return _SC_CACHE[name](*args)
    return wrapper


def _deg_body(dstq, out, dst_v, ones_v, zb, acc):
    c = lax.axis_index("c")
    s = lax.axis_index("s")
    w = c * NS + s
    base = s * RPT
    pltpu.sync_copy(dstq.at[pl.ds(w * BPW, BPW)], dst_v)
    one = jnp.ones((16,), F32)
    zero = jnp.zeros((16,), F32)
    for r in range(16):
        zb[r] = zero
    for r in range(LB):
        ones_v[r] = one

    def ib(j, carry):
        pltpu.sync_copy(zb, acc.at[pl.ds(base + j * 16, 16)])
        return carry

    lax.fori_loop(0, RPT // 16, ib, 0)
    plsc.subcore_barrier()

    def sb(j, carry):
        pltpu.sync_copy(ones_v, acc.at[dst_v.at[j]], add=True)
        return carry

    lax.fori_loop(0, BPW, sb, 0)
    plsc.subcore_barrier()

    def ob(j, carry):
        pltpu.sync_copy(acc.at[pl.ds(base + j * LB, LB)], ones_v)
        pltpu.sync_copy(ones_v, out.at[c, pl.ds(base + j * LB, LB)])
        return carry

    lax.fori_loop(0, RPT // LB, ob, 0)


def _build_deg():
    return functools.partial(
        pl.kernel,
        out_type=jax.ShapeDtypeStruct((NC, NP, 16), F32),
        mesh=_mesh(),
        scratch_types=[
            pltpu.VMEM((BPW, LB), jnp.int32),
            pltpu.VMEM((LB, 16), F32),
            pltpu.VMEM((16, 16), F32),
            pltpu.VMEM_SHARED((NP, 16), F32),
        ],
    )(_deg_body)


_deg_kernel = _lazy("deg", _build_deg)


def _make_agg(ch):
    NBUF = 2
    ZR = 32
    NGRP = BPW // NBUF

    def body(srcq, dstq, *rest):
        gs = rest[:ch]
        ps = rest[ch:2 * ch]
        rest = rest[2 * ch:]
        src_v, dst_v = rest[0], rest[1]
        rows = rest[2:2 + NBUF]
        zb, acc = rest[2 + NBUF], rest[3 + NBUF]
        sems = rest[4 + NBUF:4 + 2 * NBUF]
        c = lax.axis_index("c")
        s = lax.axis_index("s")
        w = c * NS + s
        base = s * RPT
        pltpu.sync_copy(srcq.at[pl.ds(w * BPW, BPW)], src_v)
        pltpu.sync_copy(dstq.at[pl.ds(w * BPW, BPW)], dst_v)
        zero = jnp.zeros((16,), F32)

        def zf(r, carry):
            for q in range(8):
                zb[r, pl.ds(q * 16, 16)] = zero
            return carry

        lax.fori_loop(0, ZR, zf, 0)

        def ib(j, carry):
            pltpu.sync_copy(zb, acc.at[pl.ds(base + j * ZR, ZR)])
            return carry

        lax.fori_loop(0, RPT // ZR, ib, 0)
        for fc in range(ch):
            plsc.subcore_barrier()

            for k in range(NBUF):
                pltpu.async_copy(gs[fc].at[src_v.at[k]], rows[k], sems[k])

            def grp(jj, carry, fc=fc):
                for k in range(NBUF):
                    j = jj * NBUF + k
                    pltpu.make_async_copy(
                        gs[fc].at[src_v.at[j]], rows[k], sems[k]).wait()
                    pltpu.sync_copy(rows[k], acc.at[dst_v.at[j]], add=True)

                    @pl.when(jj < NGRP - 1)
                    def _():
                        pltpu.async_copy(
                            gs[fc].at[src_v.at[j + NBUF]], rows[k], sems[k])
                return carry

            lax.fori_loop(0, NGRP, grp, 0)
            plsc.subcore_barrier()

            last = fc + 1 == ch
            for jj in range(RPT // LB):
                k = jj % NBUF
                pltpu.sync_copy(acc.at[pl.ds(base + jj * LB, LB)], rows[k])
                pltpu.sync_copy(rows[k], ps[fc].at[c, pl.ds(base + jj * LB, LB)])
                if not last:
                    for q in range(LB // ZR):
                        pltpu.sync_copy(
                            zb, acc.at[pl.ds(base + jj * LB + q * ZR, ZR)])

    def build():
        return functools.partial(
            pl.kernel,
            out_type=tuple(jax.ShapeDtypeStruct((NC, NP, 128), F32)
                           for _ in range(ch)),
            mesh=_mesh(),
            scratch_types=(
                [pltpu.VMEM((BPW, LB), jnp.int32),
                 pltpu.VMEM((BPW, LB), jnp.int32)]
                + [pltpu.VMEM((LB, 128), F32) for _ in range(NBUF)]
                + [pltpu.VMEM((ZR, 128), F32),
                   pltpu.VMEM_SHARED((NP, 128), F32)]
                + [pltpu.SemaphoreType.DMA for _ in range(NBUF)]
            ),
        )(body)

    return _lazy("agg%d" % ch, build)


_agg2 = _make_agg(2)
_agg4 = _make_agg(4)


_DOT = dict(precision=lax.Precision.HIGHEST, preferred_element_type=F32)


def _prep_body(x_ref, mean_ref, scale_ref, degp_ref, g0_ref, g1_ref, dv_ref):
    deg = degp_ref[0][:, :1] + degp_ref[1][:, :1] + 1.0
    dv = lax.rsqrt(deg)
    xt = (x_ref[...] - mean_ref[...]) / scale_ref[...] * dv
    g0_ref[...] = xt[:, :128]
    g1_ref[...] = xt[:, 128:]
    dv_ref[...] = jnp.broadcast_to(dv, (dv.shape[0], 128))


def _prep(x_pad, mean, scale, degp):
    blk = lambda shp: pl.BlockSpec(shp, lambda i: (0,) * (len(shp) - 2) + (i, 0))
    fixed = lambda shp: pl.BlockSpec(shp, lambda i: (0,) * len(shp))
    return pl.pallas_call(
        _prep_body,
        grid=(NP // BM,),
        in_specs=[
            pl.BlockSpec((BM, 256), lambda i: (i, 0)),
            fixed((1, 256)),
            fixed((1, 256)),
            pl.BlockSpec((NC, BM, 16), lambda i: (0, i, 0)),
        ],
        out_specs=[pl.BlockSpec((BM, 128), lambda i: (i, 0))] * 3,
        out_shape=[jax.ShapeDtypeStruct((NP, 128), F32)] * 3,
    )(x_pad, mean, scale, degp)


def _make_layer(ch):
    def body(*refs):
        grefs = refs[:ch]
        prefs = refs[ch:2 * ch]
        w_ref, b_ref, dv_ref = refs[2 * ch:2 * ch + 3]
        orefs = refs[2 * ch + 3:]
        u = jnp.concatenate(
            [g[...] + p[0] + p[1] for g, p in zip(grefs, prefs)], axis=1)
        z = lax.dot_general(u, w_ref[...], (((1,), (0,)), ((), ())), **_DOT)
        dv = dv_ref[...][:, :1]
        z = jnp.maximum(z * (dv * dv) + dv * b_ref[...], 0.0)
        for q in range(4):
            orefs[q][...] = z[:, 128 * q:128 * (q + 1)]

    def call(gs, ps, W, b, dvrep):
        K = ch * 128
        fixed = lambda shp: pl.BlockSpec(shp, lambda i: (0,) * len(shp))
        return pl.pallas_call(
            body,
            grid=(NP // BM,),
            in_specs=(
                [pl.BlockSpec((BM, 128), lambda i: (i, 0))] * ch
                + [pl.BlockSpec((NC, BM, 128), lambda i: (0, i, 0))] * ch
                + [fixed((K, HID)), fixed((1, HID)),
                   pl.BlockSpec((BM, 128), lambda i: (i, 0))]
            ),
            out_specs=[pl.BlockSpec((BM, 128), lambda i: (i, 0))] * 4,
            out_shape=[jax.ShapeDtypeStruct((NP, 128), F32)] * 4,
        )(*gs, *ps, W, b, dvrep)

    return call


_layer2 = _make_layer(2)
_layer4 = _make_layer(4)


def _wfold_body(w3_ref, wl_ref, b3_ref, bl_ref, w3l_ref, bf_ref):
    w3l_ref[...] = lax.dot_general(
        w3_ref[...], wl_ref[...], (((1,), (0,)), ((), ())), **_DOT)
    bf_ref[...] = lax.dot_general(
        b3_ref[...], wl_ref[...], (((1,), (0,)), ((), ())), **_DOT) + bl_ref[...]


def _wfold(W3, W_lin, b3r, blr):
    return pl.pallas_call(
        _wfold_body,
        out_shape=[jax.ShapeDtypeStruct((HID, OUTC), F32),
                   jax.ShapeDtypeStruct((1, OUTC), F32)],
    )(W3, W_lin, b3r, blr)


def _final_body(*refs):
    grefs = refs[:4]
    prefs = refs[4:8]
    w_ref, bf_ref, dv_ref, mk_ref = refs[8:12]
    o_ref = refs[12]
    u = jnp.concatenate(
        [g[...] + p[0] + p[1] for g, p in zip(grefs, prefs)], axis=1)
    z = lax.dot_general(u, w_ref[...], (((1,), (0,)), ((), ())), **_DOT)
    dv = dv_ref[...][:, :1]
    o_ref[...] = (z * dv + bf_ref[...]) * (1.0 - mk_ref[...])


def _final(gs, ps, w3l, bf, dvrep, maskf):
    fixed = lambda shp: pl.BlockSpec(shp, lambda i: (0,) * len(shp))
    return pl.pallas_call(
        _final_body,
        grid=(NP // BM,),
        in_specs=(
            [pl.BlockSpec((BM, 128), lambda i: (i, 0))] * 4
            + [pl.BlockSpec((NC, BM, 128), lambda i: (0, i, 0))] * 4
            + [fixed((HID, OUTC)), fixed((1, OUTC)),
               pl.BlockSpec((BM, 128), lambda i: (i, 0)),
               pl.BlockSpec((BM, 128), lambda i: (i, 0))]
        ),
        out_specs=pl.BlockSpec((BM, 128), lambda i: (i, 0)),
        out_shape=jax.ShapeDtypeStruct((NP, 128), F32),
    )(*gs, *ps, w3l, bf, dvrep, maskf)


def kernel(x, edge_index, feature_mask, target_vector, scaler_mean,
           scaler_scale, W0, b0, W1, b1, W2, b2, W3, b3, W_lin, b_lin):
    src = edge_index[0].astype(jnp.int32)
    dst = edge_index[1].astype(jnp.int32)
    pad = N + jnp.arange(EP - E, dtype=jnp.int32) % (NP - N)
    srcq = jnp.concatenate([src, pad]).reshape(NB, LB)
    dstq = jnp.concatenate([dst, pad]).reshape(NB, LB)

    degp = _deg_kernel(dstq)
    x_pad = jnp.pad(x, ((0, NP - N), (0, 0)))
    g0a, g0b, dvrep = _prep(x_pad, scaler_mean.reshape(1, -1),
                            scaler_scale.reshape(1, -1), degp)

    gs = [g0a, g0b]
    Ws = [W0, W1, W2]
    bs = [b0, b1, b2]
    layers = {2: _layer2, 4: _layer4}
    aggs = {2: _agg2, 4: _agg4}
    for i in range(3):
        ps = aggs[len(gs)](srcq, dstq, *gs)
        gs = layers[len(gs)](gs, ps, Ws[i], bs[i].reshape(1, -1), dvrep)

    ps = _agg4(srcq, dstq, *gs)
    w3l, bf = _wfold(W3, W_lin, b3.reshape(1, -1), b_lin.reshape(1, -1))
    maskf = jnp.pad(feature_mask.astype(F32), ((0, NP - N), (0, 0)))
    outp = _final(gs, ps, w3l, bf, dvrep, maskf)
    return outp[:N]


# speedup vs baseline: 1.0432x; 1.0432x over previous
import functools

import jax
import jax.numpy as jnp
from jax import lax
from jax.experimental import pallas as pl
from jax.experimental.pallas import tpu as pltpu
from jax.experimental.pallas import tpu_sc as plsc

N = 10000
NP = 10240
E = 160000
HID = 512
OUTC = 128
NC = 2
NS = 16
LB = 128
NW = NC * NS
EP = ((E + NW * LB - 1) // (NW * LB)) * (NW * LB)
NB = EP // LB
BPW = NB // NW
RPT = NP // NS
F32 = jnp.float32
BM = 256


def _mesh():
    return plsc.VectorSubcoreMesh(core_axis_name="c", subcore_axis_name="s",
                                  num_cores=NC, num_subcores=NS)


_SC_CACHE = {}


def _lazy(name, builder):
    def wrapper(*args):
        if name not in _SC_CACHE:
            _SC_CACHE[name] = builder()
        return _SC_CACHE[name](*args)
    return wrapper


def _deg_body(dstq, out, dst_v, ones_v, zb, acc):
    c = lax.axis_index("c")
    s = lax.axis_index("s")
    w = c * NS + s
    base = s * RPT
    pltpu.sync_copy(dstq.at[pl.ds(w * BPW, BPW)], dst_v)
    one = jnp.ones((16,), F32)
    zero = jnp.zeros((16,), F32)
    for r in range(16):
        zb[r] = zero
    for r in range(LB):
        ones_v[r] = one

    def ib(j, carry):
        pltpu.sync_copy(zb, acc.at[pl.ds(base + j * 16, 16)])
        return carry

    lax.fori_loop(0, RPT // 16, ib, 0)
    plsc.subcore_barrier()

    def sb(j, carry):
        pltpu.sync_copy(ones_v, acc.at[dst_v.at[j]], add=True)
        return carry

    lax.fori_loop(0, BPW, sb, 0)
    plsc.subcore_barrier()

    def ob(j, carry):
        pltpu.sync_copy(acc.at[pl.ds(base + j * LB, LB)], ones_v)
        pltpu.sync_copy(ones_v, out.at[c, pl.ds(base + j * LB, LB)])
        return carry

    lax.fori_loop(0, RPT // LB, ob, 0)


def _build_deg():
    return functools.partial(
        pl.kernel,
        out_type=jax.ShapeDtypeStruct((NC, NP, 16), F32),
        mesh=_mesh(),
        scratch_types=[
            pltpu.VMEM((BPW, LB), jnp.int32),
            pltpu.VMEM((LB, 16), F32),
            pltpu.VMEM((16, 16), F32),
            pltpu.VMEM_SHARED((NP, 16), F32),
        ],
    )(_deg_body)


_deg_kernel = _lazy("deg", _build_deg)


def _make_agg(ch):
    NBUF = 2
    ZR = 32
    NGRP = BPW // NBUF

    def body(srcq, dstq, *rest):
        gs = rest[:ch]
        ps = rest[ch:2 * ch]
        rest = rest[2 * ch:]
        src_v, dst_v = rest[0], rest[1]
        rows = rest[2:2 + NBUF]
        zb, acc = rest[2 + NBUF], rest[3 + NBUF]
        sems = rest[4 + NBUF:4 + 2 * NBUF]
        c = lax.axis_index("c")
        s = lax.axis_index("s")
        w = c * NS + s
        base = s * RPT
        pltpu.sync_copy(srcq.at[pl.ds(w * BPW, BPW)], src_v)
        pltpu.sync_copy(dstq.at[pl.ds(w * BPW, BPW)], dst_v)
        zero = jnp.zeros((16,), F32)

        def zf(r, carry):
            for q in range(8):
                zb[r, pl.ds(q * 16, 16)] = zero
            return carry

        lax.fori_loop(0, ZR, zf, 0)

        def ib(j, carry):
            pltpu.sync_copy(zb, acc.at[pl.ds(base + j * ZR, ZR)])
            return carry

        lax.fori_loop(0, RPT // ZR, ib, 0)
        for fc in range(ch):
            plsc.subcore_barrier()

            for k in range(NBUF):
                pltpu.async_copy(gs[fc].at[src_v.at[k]], rows[k], sems[k])

            def grp(jj, carry, fc=fc):
                for k in range(NBUF):
                    j = jj * NBUF + k
                    pltpu.make_async_copy(
                        gs[fc].at[src_v.at[j]], rows[k], sems[k]).wait()
                    pltpu.sync_copy(rows[k], acc.at[dst_v.at[j]], add=True)

                    @pl.when(jj < NGRP - 1)
                    def _():
                        pltpu.async_copy(
                            gs[fc].at[src_v.at[j + NBUF]], rows[k], sems[k])
                return carry

            lax.fori_loop(0, NGRP, grp, 0)
            plsc.subcore_barrier()

            last = fc + 1 == ch
            for jj in range(RPT // LB):
                k = jj % NBUF
                pltpu.sync_copy(acc.at[pl.ds(base + jj * LB, LB)], rows[k])
                pltpu.sync_copy(rows[k], ps[fc].at[c, pl.ds(base + jj * LB, LB)])
                if not last:
                    for q in range(LB // ZR):
                        pltpu.sync_copy(
                            zb, acc.at[pl.ds(base + jj * LB + q * ZR, ZR)])

    def build():
        return functools.partial(
            pl.kernel,
            out_type=tuple(jax.ShapeDtypeStruct((NC, NP, 128), F32)
                           for _ in range(ch)),
            mesh=_mesh(),
            scratch_types=(
                [pltpu.VMEM((BPW, LB), jnp.int32),
                 pltpu.VMEM((BPW, LB), jnp.int32)]
                + [pltpu.VMEM((LB, 128), F32) for _ in range(NBUF)]
                + [pltpu.VMEM((ZR, 128), F32),
                   pltpu.VMEM_SHARED((NP, 128), F32)]
                + [pltpu.SemaphoreType.DMA for _ in range(NBUF)]
            ),
        )(body)

    return _lazy("agg%d" % ch, build)


_agg2 = _make_agg(2)
_agg4 = _make_agg(4)


_DOT = dict(precision=lax.Precision.DEFAULT, preferred_element_type=F32)


def _prep_body(x_ref, mean_ref, scale_ref, degp_ref, g0_ref, g1_ref, dv_ref):
    deg = degp_ref[0][:, :1] + degp_ref[1][:, :1] + 1.0
    dv = lax.rsqrt(deg)
    xt = (x_ref[...] - mean_ref[...]) / scale_ref[...] * dv
    g0_ref[...] = xt[:, :128]
    g1_ref[...] = xt[:, 128:]
    dv_ref[...] = jnp.broadcast_to(dv, (dv.shape[0], 128))


def _prep(x_pad, mean, scale, degp):
    blk = lambda shp: pl.BlockSpec(shp, lambda i: (0,) * (len(shp) - 2) + (i, 0))
    fixed = lambda shp: pl.BlockSpec(shp, lambda i: (0,) * len(shp))
    return pl.pallas_call(
        _prep_body,
        grid=(NP // BM,),
        in_specs=[
            pl.BlockSpec((BM, 256), lambda i: (i, 0)),
            fixed((1, 256)),
            fixed((1, 256)),
            pl.BlockSpec((NC, BM, 16), lambda i: (0, i, 0)),
        ],
        out_specs=[pl.BlockSpec((BM, 128), lambda i: (i, 0))] * 3,
        out_shape=[jax.ShapeDtypeStruct((NP, 128), F32)] * 3,
    )(x_pad, mean, scale, degp)


def _make_layer(ch):
    def body(*refs):
        grefs = refs[:ch]
        prefs = refs[ch:2 * ch]
        w_ref, b_ref, dv_ref = refs[2 * ch:2 * ch + 3]
        orefs = refs[2 * ch + 3:]
        u = jnp.concatenate(
            [g[...] + p[0] + p[1] for g, p in zip(grefs, prefs)], axis=1)
        z = lax.dot_general(u, w_ref[...], (((1,), (0,)), ((), ())), **_DOT)
        dv = dv_ref[...][:, :1]
        z = jnp.maximum(z * (dv * dv) + dv * b_ref[...], 0.0)
        for q in range(4):
            orefs[q][...] = z[:, 128 * q:128 * (q + 1)]

    def call(gs, ps, W, b, dvrep):
        K = ch * 128
        fixed = lambda shp: pl.BlockSpec(shp, lambda i: (0,) * len(shp))
        return pl.pallas_call(
            body,
            grid=(NP // BM,),
            in_specs=(
                [pl.BlockSpec((BM, 128), lambda i: (i, 0))] * ch
                + [pl.BlockSpec((NC, BM, 128), lambda i: (0, i, 0))] * ch
                + [fixed((K, HID)), fixed((1, HID)),
                   pl.BlockSpec((BM, 128), lambda i: (i, 0))]
            ),
            out_specs=[pl.BlockSpec((BM, 128), lambda i: (i, 0))] * 4,
            out_shape=[jax.ShapeDtypeStruct((NP, 128), F32)] * 4,
        )(*gs, *ps, W, b, dvrep)

    return call


_layer2 = _make_layer(2)
_layer4 = _make_layer(4)


def _wfold_body(w3_ref, wl_ref, b3_ref, bl_ref, w3l_ref, bf_ref):
    w3l_ref[...] = lax.dot_general(
        w3_ref[...], wl_ref[...], (((1,), (0,)), ((), ())), **_DOT)
    bf_ref[...] = lax.dot_general(
        b3_ref[...], wl_ref[...], (((1,), (0,)), ((), ())), **_DOT) + bl_ref[...]


def _wfold(W3, W_lin, b3r, blr):
    return pl.pallas_call(
        _wfold_body,
        out_shape=[jax.ShapeDtypeStruct((HID, OUTC), F32),
                   jax.ShapeDtypeStruct((1, OUTC), F32)],
    )(W3, W_lin, b3r, blr)


def _final_body(*refs):
    grefs = refs[:4]
    prefs = refs[4:8]
    w_ref, bf_ref, dv_ref, mk_ref = refs[8:12]
    o_ref = refs[12]
    u = jnp.concatenate(
        [g[...] + p[0] + p[1] for g, p in zip(grefs, prefs)], axis=1)
    z = lax.dot_general(u, w_ref[...], (((1,), (0,)), ((), ())), **_DOT)
    dv = dv_ref[...][:, :1]
    o_ref[...] = (z * dv + bf_ref[...]) * (1.0 - mk_ref[...])


def _final(gs, ps, w3l, bf, dvrep, maskf):
    fixed = lambda shp: pl.BlockSpec(shp, lambda i: (0,) * len(shp))
    return pl.pallas_call(
        _final_body,
        grid=(NP // BM,),
        in_specs=(
            [pl.BlockSpec((BM, 128), lambda i: (i, 0))] * 4
            + [pl.BlockSpec((NC, BM, 128), lambda i: (0, i, 0))] * 4
            + [fixed((HID, OUTC)), fixed((1, OUTC)),
               pl.BlockSpec((BM, 128), lambda i: (i, 0)),
               pl.BlockSpec((BM, 128), lambda i: (i, 0))]
        ),
        out_specs=pl.BlockSpec((BM, 128), lambda i: (i, 0)),
        out_shape=jax.ShapeDtypeStruct((NP, 128), F32),
    )(*gs, *ps, w3l, bf, dvrep, maskf)


def kernel(x, edge_index, feature_mask, target_vector, scaler_mean,
           scaler_scale, W0, b0, W1, b1, W2, b2, W3, b3, W_lin, b_lin):
    src = edge_index[0].astype(jnp.int32)
    dst = edge_index[1].astype(jnp.int32)
    pad = N + jnp.arange(EP - E, dtype=jnp.int32) % (NP - N)
    srcq = jnp.concatenate([src, pad]).reshape(NB, LB)
    dstq = jnp.concatenate([dst, pad]).reshape(NB, LB)

    degp = _deg_kernel(dstq)
    x_pad = jnp.pad(x, ((0, NP - N), (0, 0)))
    g0a, g0b, dvrep = _prep(x_pad, scaler_mean.reshape(1, -1),
                            scaler_scale.reshape(1, -1), degp)

    gs = [g0a, g0b]
    Ws = [W0, W1, W2]
    bs = [b0, b1, b2]
    layers = {2: _layer2, 4: _layer4}
    aggs = {2: _agg2, 4: _agg4}
    for i in range(3):
        ps = aggs[len(gs)](srcq, dstq, *gs)
        gs = layers[len(gs)](gs, ps, Ws[i], bs[i].reshape(1, -1), dvrep)

    ps = _agg4(srcq, dstq, *gs)
    w3l, bf = _wfold(W3, W_lin, b3.reshape(1, -1), b_lin.reshape(1, -1))
    maskf = jnp.pad(feature_mask.astype(F32), ((0, NP - N), (0, 0)))
    outp = _final(gs, ps, w3l, bf, dvrep, maskf)
    return outp[:N]
